# baseline (speedup 1.0000x reference)
import functools

import jax
import jax.numpy as jnp
from jax import lax
from jax.experimental import pallas as pl
from jax.experimental.pallas import tpu as pltpu
from jax.experimental.pallas import tpu_sc as plsc

N_TOKENS = 32768
N_EXPERTS = 64
TOP_K = 8
D_MODEL = 768

BM = 4096
LANES = 16
NUM_WORKERS = 32

HALF_T = N_TOKENS // 2
STEPS = HALF_T // BM
RPW = HALF_T // NUM_WORKERS


def _softmax_block(x_ref, w_ref, gates_ref):
    logits = jnp.dot(x_ref[...], w_ref[...], preferred_element_type=jnp.float32)
    m = jnp.max(logits, axis=-1, keepdims=True)
    e = jnp.exp(logits - m)
    s = jnp.sum(e, axis=-1, keepdims=True)
    gates = e / s
    gates_ref[...] = gates
    return jnp.sum(gates, axis=0, keepdims=True)


def _tc_first_body(x_ref, w_ref, gates_ref, csum_ref, acc_ref):
    i = pl.program_id(0)
    csum = _softmax_block(x_ref, w_ref, gates_ref)

    @pl.when(i == 0)
    def _():
        acc_ref[...] = csum

    @pl.when(i > 0)
    def _():
        acc_ref[...] = acc_ref[...] + csum

    @pl.when(i == pl.num_programs(0) - 1)
    def _():
        csum_ref[...] = acc_ref[...]


def _tc_second_body(x_ref, w_ref, cs_ref, gates_ref, loss_ref, acc_ref):
    i = pl.program_id(0)
    csum = _softmax_block(x_ref, w_ref, gates_ref)

    @pl.when(i == 0)
    def _():
        acc_ref[...] = csum

    @pl.when(i > 0)
    def _():
        acc_ref[...] = acc_ref[...] + csum

    @pl.when(i == pl.num_programs(0) - 1)
    def _():
        c = acc_ref[...] + cs_ref[...]
        mean = jnp.sum(c) / N_EXPERTS
        var = jnp.sum((c - mean) ** 2) / (N_EXPERTS - 1)
        loss_ref[0, 0] = var / (mean + 1e-6) ** 2


def _tc_first(x, W):
    return pl.pallas_call(
        _tc_first_body,
        grid=(STEPS,),
        in_specs=[
            pl.BlockSpec((BM, D_MODEL), lambda i: (i, 0)),
            pl.BlockSpec((D_MODEL, N_EXPERTS), lambda i: (0, 0)),
        ],
        out_specs=[
            pl.BlockSpec((BM, N_EXPERTS), lambda i: (i, 0)),
            pl.BlockSpec((1, N_EXPERTS), lambda i: (0, 0)),
        ],
        out_shape=[
            jax.ShapeDtypeStruct((HALF_T, N_EXPERTS), jnp.float32),
            jax.ShapeDtypeStruct((1, N_EXPERTS), jnp.float32),
        ],
        scratch_shapes=[pltpu.VMEM((1, N_EXPERTS), jnp.float32)],
    )(x, W)


def _tc_second(x, W, cs1):
    return pl.pallas_call(
        _tc_second_body,
        grid=(STEPS,),
        in_specs=[
            pl.BlockSpec((BM, D_MODEL), lambda i: (STEPS + i, 0)),
            pl.BlockSpec((D_MODEL, N_EXPERTS), lambda i: (0, 0)),
            pl.BlockSpec((1, N_EXPERTS), lambda i: (0, 0)),
        ],
        out_specs=[
            pl.BlockSpec((BM, N_EXPERTS), lambda i: (i, 0)),
            pl.BlockSpec(memory_space=pltpu.SMEM),
        ],
        out_shape=[
            jax.ShapeDtypeStruct((HALF_T, N_EXPERTS), jnp.float32),
            jax.ShapeDtypeStruct((1, 1), jnp.float32),
        ],
        scratch_shapes=[pltpu.VMEM((1, N_EXPERTS), jnp.float32)],
    )(x, W, cs1)


def _merge_top(ka, va, kb, vb, *, descending):
    take_a = ka >= kb
    mk = jnp.where(take_a, ka, kb)
    mv = jnp.where(take_a, va, vb)
    return plsc.sort_key_val(mk, mv, descending=descending)


def _topk_row(g_v, off, iota):
    ks, vs = [], []
    for c in range(N_EXPERTS // LANES):
        k = g_v[pl.ds(pl.multiple_of(off + c * LANES, LANES), LANES)]
        sk, sv = plsc.sort_key_val(k, iota + c * LANES, descending=(c % 2 == 0))
        ks.append(sk)
        vs.append(sv)
    k01, v01 = _merge_top(ks[0], vs[0], ks[1], vs[1], descending=True)
    k23, v23 = _merge_top(ks[2], vs[2], ks[3], vs[3], descending=False)
    return _merge_top(k01, v01, k23, v23, descending=True)


def _combine_softmax(kf, lo8):
    mx = jnp.max(kf)
    e = jnp.exp(kf - mx)
    e8 = jnp.where(lo8, e, 0.0)
    s = jnp.sum(e8)
    return e8 / s


def _topk_slab(gates_hbm, src_row0, g_v, c_v, i_v):
    pltpu.sync_copy(
        gates_hbm.at[pl.ds(src_row0 * N_EXPERTS, RPW * N_EXPERTS)], g_v)

    iota = lax.iota(jnp.int32, LANES)
    lo8 = iota < TOP_K
    shift_idx = jnp.where(lo8, iota, iota - TOP_K)

    @plsc.parallel_loop(0, RPW // 2, 1, unroll=4)
    def pair_body(p):
        off0 = p * (2 * N_EXPERTS)
        k0, v0 = _topk_row(g_v, off0, iota)
        k1, v1 = _topk_row(g_v, off0 + N_EXPERTS, iota)
        c0 = _combine_softmax(k0, lo8)
        c1 = _combine_softmax(k1, lo8)
        c1s = c1.at[shift_idx].get(mode="promise_in_bounds")
        v1s = v1.at[shift_idx].get(mode="promise_in_bounds")
        cpair = jnp.where(lo8, c0, c1s)
        vpair = jnp.where(lo8, v0, v1s)
        o = pl.multiple_of(p * (2 * TOP_K), LANES)
        c_v[pl.ds(o, LANES)] = cpair
        i_v[pl.ds(o, LANES)] = vpair


def _sc_first_body(gates_hbm, comb_hbm, idx_hbm, g_v, c_v, i_v):
    wid = lax.axis_index("s") * 2 + lax.axis_index("c")
    _topk_slab(gates_hbm, wid * RPW, g_v, c_v, i_v)
    pltpu.sync_copy(c_v, comb_hbm.at[pl.ds(wid * RPW * TOP_K, RPW * TOP_K)])
    pltpu.sync_copy(i_v, idx_hbm.at[pl.ds(wid * RPW * TOP_K, RPW * TOP_K)])


def _sc_second_body(gates_hbm, comb1_hbm, idx1_hbm, comb_hbm, idx_hbm,
                    g_v, c_v, i_v, c1_v, i1_v):
    wid = lax.axis_index("s") * 2 + lax.axis_index("c")
    obase = wid * RPW * TOP_K
    _topk_slab(gates_hbm, wid * RPW, g_v, c_v, i_v)
    pltpu.sync_copy(c_v, comb_hbm.at[pl.ds(HALF_T * TOP_K + obase, RPW * TOP_K)])
    pltpu.sync_copy(i_v, idx_hbm.at[pl.ds(HALF_T * TOP_K + obase, RPW * TOP_K)])
    pltpu.sync_copy(comb1_hbm.at[pl.ds(obase, RPW * TOP_K)], c1_v)
    pltpu.sync_copy(idx1_hbm.at[pl.ds(obase, RPW * TOP_K)], i1_v)
    pltpu.sync_copy(c1_v, comb_hbm.at[pl.ds(obase, RPW * TOP_K)])
    pltpu.sync_copy(i1_v, idx_hbm.at[pl.ds(obase, RPW * TOP_K)])


@functools.cache
def _sc_first():
    return pl.kernel(
        _sc_first_body,
        out_type=(
            jax.ShapeDtypeStruct((HALF_T * TOP_K,), jnp.float32),
            jax.ShapeDtypeStruct((HALF_T * TOP_K,), jnp.int32),
        ),
        mesh=plsc.VectorSubcoreMesh(core_axis_name="c", subcore_axis_name="s"),
        compiler_params=pltpu.CompilerParams(needs_layout_passes=False),
        scratch_types=[
            pltpu.VMEM((RPW * N_EXPERTS,), jnp.float32),
            pltpu.VMEM((RPW * TOP_K,), jnp.float32),
            pltpu.VMEM((RPW * TOP_K,), jnp.int32),
        ],
    )


@functools.cache
def _sc_second():
    return pl.kernel(
        _sc_second_body,
        out_type=(
            jax.ShapeDtypeStruct((N_TOKENS * TOP_K,), jnp.float32),
            jax.ShapeDtypeStruct((N_TOKENS * TOP_K,), jnp.int32),
        ),
        mesh=plsc.VectorSubcoreMesh(core_axis_name="c", subcore_axis_name="s"),
        compiler_params=pltpu.CompilerParams(needs_layout_passes=False),
        scratch_types=[
            pltpu.VMEM((RPW * N_EXPERTS,), jnp.float32),
            pltpu.VMEM((RPW * TOP_K,), jnp.float32),
            pltpu.VMEM((RPW * TOP_K,), jnp.int32),
            pltpu.VMEM((RPW * TOP_K,), jnp.float32),
            pltpu.VMEM((RPW * TOP_K,), jnp.int32),
        ],
    )


def kernel(x, W):
    g1, cs1 = _tc_first(x, W)
    comb1, idx1 = _sc_first()(g1.reshape(-1))
    g2, loss = _tc_second(x, W, cs1)
    comb_flat, idx_flat = _sc_second()(g2.reshape(-1), comb1, idx1)
    combine_weights = comb_flat.reshape(N_TOKENS, TOP_K)
    top_k_indices = idx_flat.reshape(N_TOKENS, TOP_K)
    return combine_weights, top_k_indices, loss[0, 0]


# speedup vs baseline: 1.0372x; 1.0372x over previous
import functools

import jax
import jax.numpy as jnp
from jax import lax
from jax.experimental import pallas as pl
from jax.experimental.pallas import tpu as pltpu
from jax.experimental.pallas import tpu_sc as plsc

N_TOKENS = 32768
N_EXPERTS = 64
TOP_K = 8
D_MODEL = 768

BM = 4096
LANES = 16
NUM_WORKERS = 32
ROWS_PER_WORKER = N_TOKENS // NUM_WORKERS


def _tc_body(x_ref, w_ref, gates_ref, loss_ref, acc_ref):
    i = pl.program_id(0)
    logits = jnp.dot(x_ref[...], w_ref[...], preferred_element_type=jnp.float32)
    m = jnp.max(logits, axis=-1, keepdims=True)
    e = jnp.exp(logits - m)
    s = jnp.sum(e, axis=-1, keepdims=True)
    gates = e / s
    gates_ref[...] = gates
    csum = jnp.sum(gates, axis=0, keepdims=True)

    @pl.when(i == 0)
    def _():
        acc_ref[...] = csum

    @pl.when(i > 0)
    def _():
        acc_ref[...] = acc_ref[...] + csum

    @pl.when(i == pl.num_programs(0) - 1)
    def _():
        c = acc_ref[...]
        mean = jnp.sum(c) / N_EXPERTS
        var = jnp.sum((c - mean) ** 2) / (N_EXPERTS - 1)
        loss_ref[0, 0] = var / (mean + 1e-6) ** 2


def _tc_gates(x, W):
    return pl.pallas_call(
        _tc_body,
        grid=(N_TOKENS // BM,),
        in_specs=[
            pl.BlockSpec((BM, D_MODEL), lambda i: (i, 0)),
            pl.BlockSpec((D_MODEL, N_EXPERTS), lambda i: (0, 0)),
        ],
        out_specs=[
            pl.BlockSpec((BM, N_EXPERTS), lambda i: (i, 0)),
            pl.BlockSpec(memory_space=pltpu.SMEM),
        ],
        out_shape=[
            jax.ShapeDtypeStruct((N_TOKENS, N_EXPERTS), jnp.float32),
            jax.ShapeDtypeStruct((1, 1), jnp.float32),
        ],
        scratch_shapes=[pltpu.VMEM((1, N_EXPERTS), jnp.float32)],
    )(x, W)


def _merge_top(ka, va, kb, vb, *, descending):
    take_a = ka >= kb
    mk = jnp.where(take_a, ka, kb)
    mv = jnp.where(take_a, va, vb)
    return plsc.sort_key_val(mk, mv, descending=descending)


def _topk_row(g_v, off, iota):
    ks, vs = [], []
    for c in range(N_EXPERTS // LANES):
        k = g_v[pl.ds(pl.multiple_of(off + c * LANES, LANES), LANES)]
        sk, sv = plsc.sort_key_val(k, iota + c * LANES, descending=(c % 2 == 0))
        ks.append(sk)
        vs.append(sv)
    k01, v01 = _merge_top(ks[0], vs[0], ks[1], vs[1], descending=True)
    k23, v23 = _merge_top(ks[2], vs[2], ks[3], vs[3], descending=False)
    return _merge_top(k01, v01, k23, v23, descending=True)


def _combine_softmax(kf, lo8, zidx):
    mx = kf.at[zidx].get(mode="promise_in_bounds")
    e = jnp.exp(kf - mx)
    e8 = jnp.where(lo8, e, 0.0)
    s = jnp.sum(e8)
    return e8 / s


def _sc_body(gates_hbm, comb_hbm, idx_hbm, g_v, c_v, i_v):
    wid = lax.axis_index("s") * 2 + lax.axis_index("c")
    base = wid * ROWS_PER_WORKER
    pltpu.sync_copy(
        gates_hbm.at[pl.ds(base * N_EXPERTS, ROWS_PER_WORKER * N_EXPERTS)], g_v)

    iota = lax.iota(jnp.int32, LANES)
    lo8 = iota < TOP_K
    shift_idx = jnp.where(lo8, iota, iota - TOP_K)
    zidx = iota * 0

    @plsc.parallel_loop(0, ROWS_PER_WORKER // 2, 1, unroll=4)
    def pair_body(p):
        off0 = p * (2 * N_EXPERTS)
        k0, v0 = _topk_row(g_v, off0, iota)
        k1, v1 = _topk_row(g_v, off0 + N_EXPERTS, iota)
        c0 = _combine_softmax(k0, lo8, zidx)
        c1 = _combine_softmax(k1, lo8, zidx)
        c1s = c1.at[shift_idx].get(mode="promise_in_bounds")
        v1s = v1.at[shift_idx].get(mode="promise_in_bounds")
        cpair = jnp.where(lo8, c0, c1s)
        vpair = jnp.where(lo8, v0, v1s)
        o = pl.multiple_of(p * (2 * TOP_K), LANES)
        c_v[pl.ds(o, LANES)] = cpair
        i_v[pl.ds(o, LANES)] = vpair

    pltpu.sync_copy(c_v, comb_hbm.at[pl.ds(base * TOP_K, ROWS_PER_WORKER * TOP_K)])
    pltpu.sync_copy(i_v, idx_hbm.at[pl.ds(base * TOP_K, ROWS_PER_WORKER * TOP_K)])


@functools.cache
def _sc_topk():
    return pl.kernel(
        _sc_body,
        out_type=(
            jax.ShapeDtypeStruct((N_TOKENS * TOP_K,), jnp.float32),
            jax.ShapeDtypeStruct((N_TOKENS * TOP_K,), jnp.int32),
        ),
        mesh=plsc.VectorSubcoreMesh(core_axis_name="c", subcore_axis_name="s"),
        compiler_params=pltpu.CompilerParams(needs_layout_passes=False),
        scratch_types=[
            pltpu.VMEM((ROWS_PER_WORKER * N_EXPERTS,), jnp.float32),
            pltpu.VMEM((ROWS_PER_WORKER * TOP_K,), jnp.float32),
            pltpu.VMEM((ROWS_PER_WORKER * TOP_K,), jnp.int32),
        ],
    )


def kernel(x, W):
    gates, loss = _tc_gates(x, W)
    comb_flat, idx_flat = _sc_topk()(gates.reshape(-1))
    combine_weights = comb_flat.reshape(N_TOKENS, TOP_K)
    top_k_indices = idx_flat.reshape(N_TOKENS, TOP_K)
    return combine_weights, top_k_indices, loss[0, 0]
